# baseline (speedup 1.0000x reference)
import functools

import jax
import jax.numpy as jnp
from jax import lax
from jax.experimental import pallas as pl
from jax.experimental.pallas import tpu as pltpu
from jax.experimental.pallas import tpu_sc as plsc

TARGET = 98304
NROWS = 65536
NUP = TARGET - NROWS
D = 128
L = 16

NC, NS = 2, 16
NW = NC * NS
UP_PER_W = NUP // NW
CP_PER_W = NROWS // NW
GB = 128
NB = UP_PER_W // GB

_MESH = plsc.VectorSubcoreMesh(
    core_axis_name="c", subcore_axis_name="s", num_cores=NC, num_subcores=NS
)


@functools.partial(
    pl.kernel,
    out_type=jax.ShapeDtypeStruct((TARGET, D), jnp.float32),
    mesh=_MESH,
    scratch_types=[
        pltpu.VMEM((UP_PER_W,), jnp.int32),
        pltpu.VMEM((UP_PER_W,), jnp.int32),
        [pltpu.VMEM((GB, D), jnp.float32) for _ in range(2)],
        [pltpu.VMEM((GB, D), jnp.float32) for _ in range(2)],
        [pltpu.VMEM((GB, D), jnp.float32) for _ in range(2)],
        pltpu.SemaphoreType.DMA,
        [pltpu.SemaphoreType.DMA for _ in range(2)],
        [pltpu.SemaphoreType.DMA for _ in range(2)],
    ],
)
def _hex_unpool(x_hbm, idx0_hbm, idx1_hbm, out_hbm, i0v, i1v, r0, r1, ob, scp, sg, so):
    wid = lax.axis_index("s") * NC + lax.axis_index("c")

    cbase = wid * CP_PER_W
    cp = pltpu.async_copy(
        x_hbm.at[pl.ds(cbase, CP_PER_W)], out_hbm.at[pl.ds(cbase, CP_PER_W)], scp
    )

    ubase = wid * UP_PER_W
    pltpu.sync_copy(idx0_hbm.at[pl.ds(ubase, UP_PER_W)], i0v)
    pltpu.sync_copy(idx1_hbm.at[pl.ds(ubase, UP_PER_W)], i1v)

    def start_gathers(j, b):
        isl = pl.ds(j * GB, GB)
        d0 = pltpu.async_copy(x_hbm.at[i0v.at[isl]], r0[b], sg[b])
        d1 = pltpu.async_copy(x_hbm.at[i1v.at[isl]], r1[b], sg[b])
        return d0, d1

    pend = start_gathers(0, 0)
    stores = [None, None]
    for j in range(NB):
        b = j % 2
        nxt = None
        if j + 1 < NB:
            nxt = start_gathers(j + 1, 1 - b)
        pend[0].wait()
        pend[1].wait()
        pend = nxt

        if stores[b] is not None:
            stores[b].wait()

        def avg_body(r, carry, _b=b):
            for c in range(D // L):
                a = r0[_b][r, pl.ds(c * L, L)]
                bb = r1[_b][r, pl.ds(c * L, L)]
                ob[_b][r, pl.ds(c * L, L)] = (a + bb) * 0.5
            return carry

        lax.fori_loop(0, GB, avg_body, 0)
        stores[b] = pltpu.async_copy(
            ob[b], out_hbm.at[pl.ds(NROWS + ubase + j * GB, GB)], so[b]
        )

    for st in stores:
        if st is not None:
            st.wait()
    cp.wait()


def kernel(x, upsample_indices):
    idx0 = upsample_indices[:, 0]
    idx1 = upsample_indices[:, 1]
    return _hex_unpool(x, idx0, idx1)


# speedup vs baseline: 15.6941x; 15.6941x over previous
import functools

import jax
import jax.numpy as jnp
from jax import lax
from jax.experimental import pallas as pl
from jax.experimental.pallas import tpu as pltpu
from jax.experimental.pallas import tpu_sc as plsc

TARGET = 98304
NROWS = 65536
NUP = TARGET - NROWS
D = 128
L = 16

NC, NS = 2, 16
NW = NC * NS
UP_PER_W = NUP // NW
CP_PER_W = NROWS // NW
GB = 128
NB = UP_PER_W // GB

_MESH = plsc.VectorSubcoreMesh(
    core_axis_name="c", subcore_axis_name="s", num_cores=NC, num_subcores=NS
)


@functools.partial(
    pl.kernel,
    out_type=jax.ShapeDtypeStruct((TARGET, D), jnp.float32),
    mesh=_MESH,
    scratch_types=[
        pltpu.VMEM((UP_PER_W,), jnp.int32),
        pltpu.VMEM((UP_PER_W,), jnp.int32),
        [pltpu.VMEM((GB, D), jnp.float32) for _ in range(2)],
        [pltpu.VMEM((GB, D), jnp.float32) for _ in range(2)],
        [pltpu.VMEM((GB, D), jnp.float32) for _ in range(2)],
        [pltpu.SemaphoreType.DMA for _ in range(2)],
        [pltpu.SemaphoreType.DMA for _ in range(2)],
    ],
)
def _hex_unpool(x_hbm, idx0_hbm, idx1_hbm, out_hbm, i0v, i1v, r0, r1, ob, sg, so):
    wid = lax.axis_index("s") * NC + lax.axis_index("c")

    cbase = wid * CP_PER_W
    nchunks = CP_PER_W // GB
    bufs = [r0[0], r0[1], r1[0], r1[1]]
    sems = [sg[0], sg[1], so[0], so[1]]
    loads = [None] * 4
    stores = [None] * 4
    for b in range(4):
        loads[b] = pltpu.async_copy(x_hbm.at[pl.ds(cbase + b * GB, GB)], bufs[b], sems[b])
    for c in range(nchunks):
        b = c % 4
        loads[b].wait()
        stores[b] = pltpu.async_copy(bufs[b], out_hbm.at[pl.ds(cbase + c * GB, GB)], sems[b])
        if c + 4 < nchunks:
            stores[b].wait()
            loads[b] = pltpu.async_copy(
                x_hbm.at[pl.ds(cbase + (c + 4) * GB, GB)], bufs[b], sems[b]
            )
            stores[b] = None
    for st in stores:
        if st is not None:
            st.wait()

    ubase = wid * UP_PER_W
    pltpu.sync_copy(idx0_hbm.at[pl.ds(ubase, UP_PER_W)], i0v)
    pltpu.sync_copy(idx1_hbm.at[pl.ds(ubase, UP_PER_W)], i1v)

    def start_gathers(j, b):
        isl = pl.ds(j * GB, GB)
        d0 = pltpu.async_copy(x_hbm.at[i0v.at[isl]], r0[b], sg[b])
        d1 = pltpu.async_copy(x_hbm.at[i1v.at[isl]], r1[b], sg[b])
        return d0, d1

    pend = start_gathers(0, 0)
    outst = [None, None]
    for j in range(NB):
        b = j % 2
        nxt = None
        if j + 1 < NB:
            nxt = start_gathers(j + 1, 1 - b)
        pend[0].wait()
        pend[1].wait()
        pend = nxt

        if outst[b] is not None:
            outst[b].wait()

        def avg_body(r, carry, _b=b):
            for c in range(D // L):
                a = r0[_b][r, pl.ds(c * L, L)]
                bb = r1[_b][r, pl.ds(c * L, L)]
                ob[_b][r, pl.ds(c * L, L)] = (a + bb) * 0.5
            return carry

        lax.fori_loop(0, GB, avg_body, 0)
        outst[b] = pltpu.async_copy(
            ob[b], out_hbm.at[pl.ds(NROWS + ubase + j * GB, GB)], so[b]
        )

    for st in outst:
        if st is not None:
            st.wait()


def kernel(x, upsample_indices):
    idx0 = upsample_indices[:, 0]
    idx1 = upsample_indices[:, 1]
    return _hex_unpool(x, idx0, idx1)
